# baseline (speedup 1.0000x reference)
import functools

import jax
import jax.numpy as jnp
from jax import lax
from jax.experimental import pallas as pl
from jax.experimental.pallas import tpu as pltpu
from jax.experimental.pallas import tpu_sc as plsc

N = 10000
E = 320000
D = 128
FFN = 512
BOT = 128
INPUT = 10000

NC = 2
NS = 16
L = 16
NW = NC * NS
C = 128
NCHUNK = 80
EPAD = NW * NCHUNK * C
DCHUNK = EPAD // NS // C
NPAD = 10240
RPT = NPAD // NS
RPW = NPAD // NW

_HI = jax.lax.Precision.HIGHEST

_mesh = plsc.VectorSubcoreMesh(core_axis_name="c", subcore_axis_name="s")


@functools.partial(
    pl.kernel,
    mesh=_mesh,
    out_type=jax.ShapeDtypeStruct((NPAD,), jnp.float32),
    scratch_types=[
        pltpu.VMEM_SHARED((NPAD,), jnp.float32),
        pltpu.VMEM((DCHUNK, C), jnp.int32),
        pltpu.VMEM((DCHUNK, C), jnp.float32),
        pltpu.VMEM((RPT,), jnp.float32),
        pltpu.VMEM((RPW,), jnp.float32),
    ],
)
def _dinv_sc(dst_hbm, ew_hbm, out_hbm, deg_sh, dst_v, ew_v, zbuf_v, dbuf_v):
    cid = lax.axis_index("c")
    sid = lax.axis_index("s")

    zv = jnp.zeros((L,), jnp.float32)

    def _z(i, _):
        zbuf_v[pl.ds(i * L, L)] = zv
        return ()

    lax.fori_loop(0, RPT // L, _z, ())
    pltpu.sync_copy(zbuf_v, deg_sh.at[pl.ds(sid * RPT, RPT)])
    pltpu.sync_copy(dst_hbm.at[sid], dst_v)
    pltpu.sync_copy(ew_hbm.at[sid], ew_v)
    plsc.subcore_barrier()

    def _chunk(c, _):
        pltpu.sync_copy(ew_v.at[c], deg_sh.at[dst_v.at[c]], add=True)
        return ()

    lax.fori_loop(0, DCHUNK, _chunk, ())
    plsc.subcore_barrier()

    base = sid * RPT + cid * RPW
    pltpu.sync_copy(deg_sh.at[pl.ds(base, RPW)], dbuf_v)

    def _rs(k, _):
        d = dbuf_v[pl.ds(k * L, L)] + 1.0
        i = lax.bitcast_convert_type(d, jnp.int32)
        i = jnp.int32(0x5F3759DF) - lax.shift_right_arithmetic(i, 1)
        y = lax.bitcast_convert_type(i, jnp.float32)
        y = y * (1.5 - 0.5 * d * y * y)
        y = y * (1.5 - 0.5 * d * y * y)
        y = y * (1.5 - 0.5 * d * y * y)
        dbuf_v[pl.ds(k * L, L)] = y
        return ()

    lax.fori_loop(0, RPW // L, _rs, ())
    pltpu.sync_copy(dbuf_v, out_hbm.at[pl.ds(base, RPW)])


@functools.partial(
    pl.kernel,
    mesh=_mesh,
    out_type=jax.ShapeDtypeStruct((NC, NPAD, D), jnp.float32),
    scratch_types=[
        pltpu.VMEM_SHARED((NPAD, D), jnp.float32),
        pltpu.VMEM((C,), jnp.int32),
        pltpu.VMEM((C,), jnp.int32),
        pltpu.VMEM((C,), jnp.int32),
        pltpu.VMEM((C,), jnp.int32),
        pltpu.VMEM((C,), jnp.float32),
        pltpu.VMEM((C,), jnp.float32),
        pltpu.VMEM((C, D), jnp.float32),
        pltpu.VMEM((C, D), jnp.float32),
        pltpu.SemaphoreType.DMA,
        pltpu.SemaphoreType.DMA,
        pltpu.SemaphoreType.DMA,
    ],
)
def _agg_sc(h_hbm, src_hbm, dst_hbm, ew_hbm, out_hbm,
            acc_sh, src0, src1, dst0, dst1, ew0, ew1, rows0, rows1,
            isem0, isem1, gsem):
    cid = lax.axis_index("c")
    sid = lax.axis_index("s")
    wid = sid * NC + cid

    zv = jnp.zeros((L,), jnp.float32)

    def _zrow(i, _):
        def _zcol(j, _):
            rows0[i, pl.ds(j * L, L)] = zv
            return ()
        lax.fori_loop(0, D // L, _zcol, ())
        return ()

    lax.fori_loop(0, C, _zrow, ())
    for k in range(RPT // C):
        pltpu.sync_copy(rows0, acc_sh.at[pl.ds(sid * RPT + k * C, C)])
    plsc.subcore_barrier()

    def _idx(c, sb, db, eb, sem):
        return (pltpu.make_async_copy(src_hbm.at[wid, c], sb, sem),
                pltpu.make_async_copy(dst_hbm.at[wid, c], db, sem),
                pltpu.make_async_copy(ew_hbm.at[wid, c], eb, sem))

    def _issue_idx(c, sb, db, eb, sem):
        for cp in _idx(c, sb, db, eb, sem):
            cp.start()

    def _wait_idx(c, sb, db, eb, sem):
        for cp in _idx(c, sb, db, eb, sem):
            cp.wait()

    def _gather(sb, rows):
        return pltpu.make_async_copy(h_hbm.at[sb], rows, gsem)

    def _scale_scatter(rows, eb, db):
        def _scale(e, _):
            w16 = eb[pl.ds(e & -L, L)]
            lane = jnp.full((L, 1), e & (L - 1), jnp.int32)
            w = lax.gather(
                w16, lane,
                lax.GatherDimensionNumbers(offset_dims=(),
                                           collapsed_slice_dims=(0,),
                                           start_index_map=(0,)),
                slice_sizes=(1,),
                mode=lax.GatherScatterMode.PROMISE_IN_BOUNDS)
            for j in range(D // L):
                rows[e, pl.ds(j * L, L)] = rows[e, pl.ds(j * L, L)] * w
            return ()

        lax.fori_loop(0, C, _scale, ())
        pltpu.sync_copy(rows, acc_sh.at[db], add=True)

    _issue_idx(0, src0, dst0, ew0, isem0)
    _wait_idx(0, src0, dst0, ew0, isem0)
    _gather(src0, rows0).start()
    _issue_idx(1, src1, dst1, ew1, isem1)

    def _pair(g, _):
        a = 2 * g
        _wait_idx(a + 1, src1, dst1, ew1, isem1)
        _gather(src0, rows0).wait()
        _gather(src1, rows1).start()
        _scale_scatter(rows0, ew0, dst0)
        _issue_idx(a + 2, src0, dst0, ew0, isem0)
        _wait_idx(a + 2, src0, dst0, ew0, isem0)
        _gather(src1, rows1).wait()
        _gather(src0, rows0).start()
        _scale_scatter(rows1, ew1, dst1)
        _issue_idx(a + 3, src1, dst1, ew1, isem1)
        return ()

    lax.fori_loop(0, NCHUNK // 2 - 1, _pair, ())
    _wait_idx(NCHUNK - 1, src1, dst1, ew1, isem1)
    _gather(src0, rows0).wait()
    _gather(src1, rows1).start()
    _scale_scatter(rows0, ew0, dst0)
    _gather(src1, rows1).wait()
    _scale_scatter(rows1, ew1, dst1)

    plsc.subcore_barrier()
    pltpu.sync_copy(acc_sh.at[pl.ds(sid * RPT, RPT)],
                    out_hbm.at[cid, pl.ds(sid * RPT, RPT)])


RB = 2000
GRID = N // RB


def _mm_body(x_ref, w_ref, o_ref):
    o_ref[...] = lax.dot_general(x_ref[...], w_ref[...],
                                 (((1,), (0,)), ((), ())),
                                 preferred_element_type=jnp.float32,
                                 precision=_HI)


def _mm(x, w):
    return pl.pallas_call(
        _mm_body,
        grid=(GRID,),
        in_specs=[pl.BlockSpec((RB, D), lambda i: (i, 0)),
                  pl.BlockSpec((D, D), lambda i: (0, 0))],
        out_specs=pl.BlockSpec((RB, D), lambda i: (i, 0)),
        out_shape=jax.ShapeDtypeStruct((N, D), jnp.float32),
    )(x, w)


def _prep_body(h_ref, dinv_ref, hs_ref):
    hs_ref[...] = h_ref[...] * dinv_ref[...]


def _prep(h, dinv):
    return pl.pallas_call(
        _prep_body,
        grid=(GRID,),
        in_specs=[pl.BlockSpec((RB, D), lambda i: (i, 0)),
                  pl.BlockSpec((RB, 1), lambda i: (i, 0))],
        out_specs=pl.BlockSpec((RB, D), lambda i: (i, 0)),
        out_shape=jax.ShapeDtypeStruct((N, D), jnp.float32),
    )(h, dinv)


def _elu(t):
    return jnp.where(t > 0, t, jnp.exp(t) - 1.0)


def _comb1_body(a0_ref, a1_ref, hs_ref, dinv_ref, b_ref, w_ref, o_ref):
    dinv = dinv_ref[...]
    t = dinv * (a0_ref[0] + a1_ref[0] + hs_ref[...]) + b_ref[...]
    t = _elu(t)
    o_ref[...] = dinv * lax.dot_general(t, w_ref[...],
                                        (((1,), (0,)), ((), ())),
                                        preferred_element_type=jnp.float32,
                                        precision=_HI)


def _comb1(acc, hs, dinv, b, w):
    return pl.pallas_call(
        _comb1_body,
        grid=(GRID,),
        in_specs=[pl.BlockSpec((1, RB, D), lambda i: (0, i, 0)),
                  pl.BlockSpec((1, RB, D), lambda i: (1, i, 0)),
                  pl.BlockSpec((RB, D), lambda i: (i, 0)),
                  pl.BlockSpec((RB, 1), lambda i: (i, 0)),
                  pl.BlockSpec((1, D), lambda i: (0, 0)),
                  pl.BlockSpec((D, D), lambda i: (0, 0))],
        out_specs=pl.BlockSpec((RB, D), lambda i: (i, 0)),
        out_shape=jax.ShapeDtypeStruct((N, D), jnp.float32),
    )(acc, acc, hs, dinv, b, w)


def _comb2_body(a0_ref, a1_ref, hs_ref, dinv_ref, b_ref, o_ref):
    t = dinv_ref[...] * (a0_ref[0] + a1_ref[0] + hs_ref[...]) + b_ref[...]
    o_ref[...] = _elu(t)


def _comb2(acc, hs, dinv, b):
    return pl.pallas_call(
        _comb2_body,
        grid=(GRID,),
        in_specs=[pl.BlockSpec((1, RB, D), lambda i: (0, i, 0)),
                  pl.BlockSpec((1, RB, D), lambda i: (1, i, 0)),
                  pl.BlockSpec((RB, D), lambda i: (i, 0)),
                  pl.BlockSpec((RB, 1), lambda i: (i, 0)),
                  pl.BlockSpec((1, D), lambda i: (0, 0))],
        out_specs=pl.BlockSpec((RB, D), lambda i: (i, 0)),
        out_shape=jax.ShapeDtypeStruct((N, D), jnp.float32),
    )(acc, acc, hs, dinv, b)


KB = 2000
KGRID = INPUT // KB


def _fc_body(rt_ref, w1_ref, b1_ref, w2_ref, b2_ref, o_ref, acc_ref):
    i = pl.program_id(0)

    @pl.when(i == 0)
    def _():
        acc_ref[...] = jnp.zeros_like(acc_ref)

    acc_ref[...] += lax.dot_general(rt_ref[...], w1_ref[...],
                                    (((0,), (0,)), ((), ())),
                                    preferred_element_type=jnp.float32,
                                    precision=_HI)

    @pl.when(i == KGRID - 1)
    def _():
        z = _elu(acc_ref[...] + b1_ref[...])
        y = lax.dot_general(z, w2_ref[...], (((1,), (0,)), ((), ())),
                            preferred_element_type=jnp.float32,
                            precision=_HI) + b2_ref[...]
        o_ref[...] = _elu(y)


def _fc(rt, w1, b1, w2, b2):
    return pl.pallas_call(
        _fc_body,
        grid=(KGRID,),
        in_specs=[pl.BlockSpec((KB, D), lambda i: (i, 0)),
                  pl.BlockSpec((KB, FFN), lambda i: (i, 0)),
                  pl.BlockSpec((1, FFN), lambda i: (0, 0)),
                  pl.BlockSpec((FFN, BOT), lambda i: (0, 0)),
                  pl.BlockSpec((1, BOT), lambda i: (0, 0))],
        out_specs=pl.BlockSpec((D, BOT), lambda i: (0, 0)),
        out_shape=jax.ShapeDtypeStruct((D, BOT), jnp.float32),
        scratch_shapes=[pltpu.VMEM((D, FFN), jnp.float32)],
        compiler_params=pltpu.CompilerParams(
            dimension_semantics=("arbitrary",)),
    )(rt, w1, b1, w2, b2)


def kernel(x, edge_index, edge_weight, W1, b1, W2, b2,
           fc1_W, fc1_b, fc2_W, fc2_b):
    pad = EPAD - E
    srcp = jnp.concatenate([edge_index[0],
                            jnp.zeros((pad,), jnp.int32)])
    dstp = jnp.concatenate([edge_index[1],
                            jnp.full((pad,), NPAD - 1, jnp.int32)])
    ewp = jnp.concatenate([edge_weight, jnp.zeros((pad,), jnp.float32)])
    src3 = srcp.reshape(NW, NCHUNK, C)
    dst3 = dstp.reshape(NW, NCHUNK, C)
    ew2 = ewp.reshape(NW, NCHUNK, C)
    dst_deg = dstp.reshape(NS, DCHUNK, C)
    ew_deg = ewp.reshape(NS, DCHUNK, C)

    dinv_vec = _dinv_sc(dst_deg, ew_deg)
    h1 = _mm(x, W1)
    dinv = dinv_vec.reshape(NPAD, 1)
    h1s = _prep(h1, dinv)
    acc1 = _agg_sc(h1s, src3, dst3, ew2)
    h2s = _comb1(acc1, h1s, dinv, b1.reshape(1, D), W2)
    acc2 = _agg_sc(h2s, src3, dst3, ew2)
    out2 = _comb2(acc2, h2s, dinv, b2.reshape(1, D))
    rt = out2.reshape(D, INPUT).T
    return _fc(rt, fc1_W, fc1_b.reshape(1, FFN), fc2_W, fc2_b.reshape(1, BOT))


# speedup vs baseline: 1.0020x; 1.0020x over previous
import functools

import jax
import jax.numpy as jnp
from jax import lax
from jax.experimental import pallas as pl
from jax.experimental.pallas import tpu as pltpu
from jax.experimental.pallas import tpu_sc as plsc

N = 10000
E = 320000
D = 128
FFN = 512
BOT = 128
INPUT = 10000

NC = 2
NS = 16
L = 16
NW = NC * NS
C = 128
NCHUNK = 80
EPAD = NW * NCHUNK * C
DCHUNK = EPAD // NS // C
NPAD = 10240
RPT = NPAD // NS
RPW = NPAD // NW

_HI = jax.lax.Precision.HIGHEST

_mesh = plsc.VectorSubcoreMesh(core_axis_name="c", subcore_axis_name="s")


@functools.partial(
    pl.kernel,
    mesh=_mesh,
    out_type=jax.ShapeDtypeStruct((NPAD,), jnp.float32),
    scratch_types=[
        pltpu.VMEM_SHARED((NPAD,), jnp.float32),
        pltpu.VMEM((DCHUNK, C), jnp.int32),
        pltpu.VMEM((DCHUNK, C), jnp.float32),
        pltpu.VMEM((RPT,), jnp.float32),
        pltpu.VMEM((RPW,), jnp.float32),
    ],
)
def _dinv_sc(dst_hbm, ew_hbm, out_hbm, deg_sh, dst_v, ew_v, zbuf_v, dbuf_v):
    cid = lax.axis_index("c")
    sid = lax.axis_index("s")

    zv = jnp.zeros((L,), jnp.float32)

    def _z(i, _):
        zbuf_v[pl.ds(i * L, L)] = zv
        return ()

    lax.fori_loop(0, RPT // L, _z, ())
    pltpu.sync_copy(zbuf_v, deg_sh.at[pl.ds(sid * RPT, RPT)])
    pltpu.sync_copy(dst_hbm.at[sid], dst_v)
    pltpu.sync_copy(ew_hbm.at[sid], ew_v)
    plsc.subcore_barrier()

    def _chunk(c, _):
        pltpu.sync_copy(ew_v.at[c], deg_sh.at[dst_v.at[c]], add=True)
        return ()

    lax.fori_loop(0, DCHUNK, _chunk, ())
    plsc.subcore_barrier()

    base = sid * RPT + cid * RPW
    pltpu.sync_copy(deg_sh.at[pl.ds(base, RPW)], dbuf_v)

    def _rs(k, _):
        d = dbuf_v[pl.ds(k * L, L)] + 1.0
        i = lax.bitcast_convert_type(d, jnp.int32)
        i = jnp.int32(0x5F3759DF) - lax.shift_right_arithmetic(i, 1)
        y = lax.bitcast_convert_type(i, jnp.float32)
        y = y * (1.5 - 0.5 * d * y * y)
        y = y * (1.5 - 0.5 * d * y * y)
        y = y * (1.5 - 0.5 * d * y * y)
        dbuf_v[pl.ds(k * L, L)] = y
        return ()

    lax.fori_loop(0, RPW // L, _rs, ())
    pltpu.sync_copy(dbuf_v, out_hbm.at[pl.ds(base, RPW)])


@functools.partial(
    pl.kernel,
    mesh=_mesh,
    out_type=jax.ShapeDtypeStruct((NC, NPAD, D), jnp.float32),
    scratch_types=[
        pltpu.VMEM_SHARED((NPAD, D), jnp.float32),
        pltpu.VMEM((C,), jnp.int32),
        pltpu.VMEM((C,), jnp.int32),
        pltpu.VMEM((C,), jnp.int32),
        pltpu.VMEM((C,), jnp.int32),
        pltpu.VMEM((C,), jnp.float32),
        pltpu.VMEM((C,), jnp.float32),
        pltpu.VMEM((C, D), jnp.float32),
        pltpu.VMEM((C, D), jnp.float32),
        pltpu.SemaphoreType.DMA,
        pltpu.SemaphoreType.DMA,
        pltpu.SemaphoreType.DMA,
    ],
)
def _agg_sc(h_hbm, src_hbm, dst_hbm, ew_hbm, out_hbm,
            acc_sh, src0, src1, dst0, dst1, ew0, ew1, rows0, rows1,
            isem0, isem1, gsem):
    cid = lax.axis_index("c")
    sid = lax.axis_index("s")
    wid = sid * NC + cid

    zv = jnp.zeros((L,), jnp.float32)

    def _zrow(i, _):
        def _zcol(j, _):
            rows0[i, pl.ds(j * L, L)] = zv
            return ()
        lax.fori_loop(0, D // L, _zcol, ())
        return ()

    lax.fori_loop(0, C, _zrow, ())
    for k in range(RPT // C):
        pltpu.sync_copy(rows0, acc_sh.at[pl.ds(sid * RPT + k * C, C)])
    plsc.subcore_barrier()

    def _idx(c, sb, db, eb, sem):
        return (pltpu.make_async_copy(src_hbm.at[wid, c], sb, sem),
                pltpu.make_async_copy(dst_hbm.at[wid, c], db, sem),
                pltpu.make_async_copy(ew_hbm.at[wid, c], eb, sem))

    def _issue_idx(c, sb, db, eb, sem):
        for cp in _idx(c, sb, db, eb, sem):
            cp.start()

    def _wait_idx(c, sb, db, eb, sem):
        for cp in _idx(c, sb, db, eb, sem):
            cp.wait()

    def _gather(sb, rows):
        return pltpu.make_async_copy(h_hbm.at[sb], rows, gsem)

    def _scale_scatter(rows, eb, db):
        def _scale(e, _):
            w16 = eb[pl.ds(e & -L, L)]
            lane = jnp.full((L, 1), e & (L - 1), jnp.int32)
            w = lax.gather(
                w16, lane,
                lax.GatherDimensionNumbers(offset_dims=(),
                                           collapsed_slice_dims=(0,),
                                           start_index_map=(0,)),
                slice_sizes=(1,),
                mode=lax.GatherScatterMode.PROMISE_IN_BOUNDS)
            for j in range(D // L):
                rows[e, pl.ds(j * L, L)] = rows[e, pl.ds(j * L, L)] * w
            return ()

        lax.fori_loop(0, C, _scale, ())
        pltpu.sync_copy(rows, acc_sh.at[db], add=True)

    _issue_idx(0, src0, dst0, ew0, isem0)
    _wait_idx(0, src0, dst0, ew0, isem0)
    _gather(src0, rows0).start()
    _issue_idx(1, src1, dst1, ew1, isem1)

    def _pair(g, _):
        a = 2 * g
        _wait_idx(a + 1, src1, dst1, ew1, isem1)
        _gather(src0, rows0).wait()
        _gather(src1, rows1).start()
        _scale_scatter(rows0, ew0, dst0)
        _issue_idx(a + 2, src0, dst0, ew0, isem0)
        _wait_idx(a + 2, src0, dst0, ew0, isem0)
        _gather(src1, rows1).wait()
        _gather(src0, rows0).start()
        _scale_scatter(rows1, ew1, dst1)
        _issue_idx(a + 3, src1, dst1, ew1, isem1)
        return ()

    lax.fori_loop(0, NCHUNK // 2 - 1, _pair, ())
    _wait_idx(NCHUNK - 1, src1, dst1, ew1, isem1)
    _gather(src0, rows0).wait()
    _gather(src1, rows1).start()
    _scale_scatter(rows0, ew0, dst0)
    _gather(src1, rows1).wait()
    _scale_scatter(rows1, ew1, dst1)

    plsc.subcore_barrier()
    pltpu.sync_copy(acc_sh.at[pl.ds(sid * RPT, RPT)],
                    out_hbm.at[cid, pl.ds(sid * RPT, RPT)])


RB = 2000
GRID = N // RB


def _mm_body(x_ref, w_ref, o_ref):
    o_ref[...] = lax.dot_general(x_ref[...], w_ref[...],
                                 (((1,), (0,)), ((), ())),
                                 preferred_element_type=jnp.float32,
                                 precision=_HI)


def _mm(x, w):
    return pl.pallas_call(
        _mm_body,
        grid=(GRID,),
        in_specs=[pl.BlockSpec((RB, D), lambda i: (i, 0)),
                  pl.BlockSpec((D, D), lambda i: (0, 0))],
        out_specs=pl.BlockSpec((RB, D), lambda i: (i, 0)),
        out_shape=jax.ShapeDtypeStruct((N, D), jnp.float32),
    )(x, w)


def _prep_body(h_ref, dinv_ref, hs_ref):
    hs_ref[...] = h_ref[...] * dinv_ref[...]


def _prep(h, dinv):
    return pl.pallas_call(
        _prep_body,
        grid=(GRID,),
        in_specs=[pl.BlockSpec((RB, D), lambda i: (i, 0)),
                  pl.BlockSpec((RB, 1), lambda i: (i, 0))],
        out_specs=pl.BlockSpec((RB, D), lambda i: (i, 0)),
        out_shape=jax.ShapeDtypeStruct((N, D), jnp.float32),
    )(h, dinv)


def _elu(t):
    return jnp.where(t > 0, t, jnp.exp(t) - 1.0)


def _comb1_body(a0_ref, a1_ref, hs_ref, dinv_ref, b_ref, w_ref, o_ref):
    dinv = dinv_ref[...]
    t = dinv * (a0_ref[0] + a1_ref[0] + hs_ref[...]) + b_ref[...]
    t = _elu(t)
    o_ref[...] = dinv * lax.dot_general(t, w_ref[...],
                                        (((1,), (0,)), ((), ())),
                                        preferred_element_type=jnp.float32,
                                        precision=_HI)


def _comb1(acc, hs, dinv, b, w):
    return pl.pallas_call(
        _comb1_body,
        grid=(GRID,),
        in_specs=[pl.BlockSpec((1, RB, D), lambda i: (0, i, 0)),
                  pl.BlockSpec((1, RB, D), lambda i: (1, i, 0)),
                  pl.BlockSpec((RB, D), lambda i: (i, 0)),
                  pl.BlockSpec((RB, 1), lambda i: (i, 0)),
                  pl.BlockSpec((1, D), lambda i: (0, 0)),
                  pl.BlockSpec((D, D), lambda i: (0, 0))],
        out_specs=pl.BlockSpec((RB, D), lambda i: (i, 0)),
        out_shape=jax.ShapeDtypeStruct((N, D), jnp.float32),
    )(acc, acc, hs, dinv, b, w)


def _comb2_body(a0_ref, a1_ref, hs_ref, dinv_ref, b_ref, o_ref):
    t = dinv_ref[...] * (a0_ref[0] + a1_ref[0] + hs_ref[...]) + b_ref[...]
    o_ref[...] = _elu(t)


def _comb2(acc, hs, dinv, b):
    return pl.pallas_call(
        _comb2_body,
        grid=(GRID,),
        in_specs=[pl.BlockSpec((1, RB, D), lambda i: (0, i, 0)),
                  pl.BlockSpec((1, RB, D), lambda i: (1, i, 0)),
                  pl.BlockSpec((RB, D), lambda i: (i, 0)),
                  pl.BlockSpec((RB, 1), lambda i: (i, 0)),
                  pl.BlockSpec((1, D), lambda i: (0, 0))],
        out_specs=pl.BlockSpec((RB, D), lambda i: (i, 0)),
        out_shape=jax.ShapeDtypeStruct((N, D), jnp.float32),
    )(acc, acc, hs, dinv, b)


KB = 2000
KGRID = INPUT // KB


def _fc_body(rt_ref, w1_ref, b1_ref, w2_ref, b2_ref, o_ref, acc_ref):
    i = pl.program_id(0)

    @pl.when(i == 0)
    def _():
        acc_ref[...] = jnp.zeros_like(acc_ref)

    acc_ref[...] += lax.dot_general(rt_ref[...], w1_ref[...],
                                    (((0,), (0,)), ((), ())),
                                    preferred_element_type=jnp.float32,
                                    precision=_HI)

    @pl.when(i == KGRID - 1)
    def _():
        z = _elu(acc_ref[...] + b1_ref[...])
        y = lax.dot_general(z, w2_ref[...], (((1,), (0,)), ((), ())),
                            preferred_element_type=jnp.float32,
                            precision=_HI) + b2_ref[...]
        o_ref[...] = _elu(y)


def _fc(rt, w1, b1, w2, b2):
    return pl.pallas_call(
        _fc_body,
        grid=(KGRID,),
        in_specs=[pl.BlockSpec((KB, D), lambda i: (i, 0)),
                  pl.BlockSpec((KB, FFN), lambda i: (i, 0)),
                  pl.BlockSpec((1, FFN), lambda i: (0, 0)),
                  pl.BlockSpec((FFN, BOT), lambda i: (0, 0)),
                  pl.BlockSpec((1, BOT), lambda i: (0, 0))],
        out_specs=pl.BlockSpec((D, BOT), lambda i: (0, 0)),
        out_shape=jax.ShapeDtypeStruct((D, BOT), jnp.float32),
        scratch_shapes=[pltpu.VMEM((D, FFN), jnp.float32)],
        compiler_params=pltpu.CompilerParams(
            dimension_semantics=("arbitrary",)),
    )(rt, w1, b1, w2, b2)


def kernel(x, edge_index, edge_weight, W1, b1, W2, b2,
           fc1_W, fc1_b, fc2_W, fc2_b):
    pad = EPAD - E
    srcp = jnp.concatenate([edge_index[0],
                            jnp.zeros((pad,), jnp.int32)])
    pad_dst = N + jax.lax.rem(jnp.arange(pad, dtype=jnp.int32),
                              jnp.int32(NPAD - N))
    dstp = jnp.concatenate([edge_index[1], pad_dst])
    ewp = jnp.concatenate([edge_weight, jnp.zeros((pad,), jnp.float32)])
    src3 = srcp.reshape(NW, NCHUNK, C)
    dst3 = dstp.reshape(NW, NCHUNK, C)
    ew2 = ewp.reshape(NW, NCHUNK, C)
    dst_deg = dstp.reshape(NS, DCHUNK, C)
    ew_deg = ewp.reshape(NS, DCHUNK, C)

    dinv_vec = _dinv_sc(dst_deg, ew_deg)
    h1 = _mm(x, W1)
    dinv = dinv_vec.reshape(NPAD, 1)
    h1s = _prep(h1, dinv)
    acc1 = _agg_sc(h1s, src3, dst3, ew2)
    h2s = _comb1(acc1, h1s, dinv, b1.reshape(1, D), W2)
    acc2 = _agg_sc(h2s, src3, dst3, ew2)
    out2 = _comb2(acc2, h2s, dinv, b2.reshape(1, D))
    rt = out2.reshape(D, INPUT).T
    return _fc(rt, fc1_W, fc1_b.reshape(1, FFN), fc2_W, fc2_b.reshape(1, BOT))


# speedup vs baseline: 1.5292x; 1.5261x over previous
import functools

import jax
import jax.numpy as jnp
from jax import lax
from jax.experimental import pallas as pl
from jax.experimental.pallas import tpu as pltpu
from jax.experimental.pallas import tpu_sc as plsc

N = 10000
E = 320000
D = 128
DH = D // 2
FFN = 512
BOT = 128
INPUT = 10000

NC = 2
NS = 16
L = 16
C = 128
DCHUNK = 160
EPAD = NS * DCHUNK * C
NPAD = 10240
RPT = NPAD // NS
RPW = NPAD // (NC * NS)

_HI = jax.lax.Precision.HIGHEST

_mesh = plsc.VectorSubcoreMesh(core_axis_name="c", subcore_axis_name="s")


@functools.partial(
    pl.kernel,
    mesh=_mesh,
    out_type=jax.ShapeDtypeStruct((NPAD,), jnp.float32),
    scratch_types=[
        pltpu.VMEM_SHARED((NPAD,), jnp.float32),
        pltpu.VMEM((DCHUNK, C), jnp.int32),
        pltpu.VMEM((DCHUNK, C), jnp.float32),
        pltpu.VMEM((RPT,), jnp.float32),
        pltpu.VMEM((RPW,), jnp.float32),
    ],
)
def _dinv_sc(dst_hbm, ew_hbm, out_hbm, deg_sh, dst_v, ew_v, zbuf_v, dbuf_v):
    cid = lax.axis_index("c")
    sid = lax.axis_index("s")

    zv = jnp.zeros((L,), jnp.float32)

    def _z(i, _):
        zbuf_v[pl.ds(i * L, L)] = zv
        return ()

    lax.fori_loop(0, RPT // L, _z, ())
    pltpu.sync_copy(zbuf_v, deg_sh.at[pl.ds(sid * RPT, RPT)])
    pltpu.sync_copy(dst_hbm.at[sid], dst_v)
    pltpu.sync_copy(ew_hbm.at[sid], ew_v)
    plsc.subcore_barrier()

    def _chunk(c, _):
        pltpu.sync_copy(ew_v.at[c], deg_sh.at[dst_v.at[c]], add=True)
        return ()

    lax.fori_loop(0, DCHUNK, _chunk, ())
    plsc.subcore_barrier()

    base = sid * RPT + cid * RPW
    pltpu.sync_copy(deg_sh.at[pl.ds(base, RPW)], dbuf_v)

    def _rs(k, _):
        d = dbuf_v[pl.ds(k * L, L)] + 1.0
        i = lax.bitcast_convert_type(d, jnp.int32)
        i = jnp.int32(0x5F3759DF) - lax.shift_right_arithmetic(i, 1)
        y = lax.bitcast_convert_type(i, jnp.float32)
        y = y * (1.5 - 0.5 * d * y * y)
        y = y * (1.5 - 0.5 * d * y * y)
        y = y * (1.5 - 0.5 * d * y * y)
        dbuf_v[pl.ds(k * L, L)] = y
        return ()

    lax.fori_loop(0, RPW // L, _rs, ())
    pltpu.sync_copy(dbuf_v, out_hbm.at[pl.ds(base, RPW)])


@functools.partial(
    pl.kernel,
    mesh=_mesh,
    out_type=jax.ShapeDtypeStruct((NC, NPAD, DH), jnp.float32),
    scratch_types=[
        pltpu.VMEM_SHARED((NPAD, DH), jnp.float32),
        pltpu.VMEM_SHARED((NPAD, DH), jnp.float32),
        pltpu.VMEM((C,), jnp.int32),
        pltpu.VMEM((C,), jnp.int32),
        pltpu.VMEM((C,), jnp.int32),
        pltpu.VMEM((C,), jnp.int32),
        pltpu.VMEM((C,), jnp.float32),
        pltpu.VMEM((C,), jnp.float32),
        pltpu.VMEM((C, DH), jnp.float32),
        pltpu.VMEM((C, DH), jnp.float32),
        pltpu.SemaphoreType.DMA,
        pltpu.SemaphoreType.DMA,
        pltpu.SemaphoreType.DMA,
        pltpu.SemaphoreType.DMA,
    ],
)
def _agg_sc(h_hbm, src_hbm, dst_hbm, ew_hbm, out_hbm,
            h_sh, acc_sh, src0, src1, dst0, dst1, ew0, ew1, rows0, rows1,
            hsem, isem0, isem1, gsem):
    cid = lax.axis_index("c")
    sid = lax.axis_index("s")

    hload = pltpu.make_async_copy(h_hbm.at[cid, pl.ds(sid * RPT, RPT)],
                                  h_sh.at[pl.ds(sid * RPT, RPT)], hsem)
    hload.start()

    zv = jnp.zeros((L,), jnp.float32)

    def _zrow(i, _):
        def _zcol(j, _):
            rows0[i, pl.ds(j * L, L)] = zv
            return ()
        lax.fori_loop(0, DH // L, _zcol, ())
        return ()

    lax.fori_loop(0, C, _zrow, ())
    for k in range(RPT // C):
        pltpu.sync_copy(rows0, acc_sh.at[pl.ds(sid * RPT + k * C, C)])
    hload.wait()
    plsc.subcore_barrier()

    def _idx(c, sb, db, eb, sem):
        return (pltpu.make_async_copy(src_hbm.at[sid, c], sb, sem),
                pltpu.make_async_copy(dst_hbm.at[sid, c], db, sem),
                pltpu.make_async_copy(ew_hbm.at[sid, c], eb, sem))

    def _issue_idx(c, sb, db, eb, sem):
        for cp in _idx(c, sb, db, eb, sem):
            cp.start()

    def _wait_idx(c, sb, db, eb, sem):
        for cp in _idx(c, sb, db, eb, sem):
            cp.wait()

    def _gather(sb, rows):
        return pltpu.make_async_copy(h_sh.at[sb], rows, gsem)

    def _scale(rows, eb):
        def _edge(e, _):
            w16 = eb[pl.ds(e & -L, L)]
            lane = jnp.full((L, 1), e & (L - 1), jnp.int32)
            w = lax.gather(
                w16, lane,
                lax.GatherDimensionNumbers(offset_dims=(),
                                           collapsed_slice_dims=(0,),
                                           start_index_map=(0,)),
                slice_sizes=(1,),
                mode=lax.GatherScatterMode.PROMISE_IN_BOUNDS)
            for j in range(DH // L):
                rows[e, pl.ds(j * L, L)] = rows[e, pl.ds(j * L, L)] * w
            return ()

        lax.fori_loop(0, C, _edge, ())

    def _scatter(rows, db):
        pltpu.sync_copy(rows, acc_sh.at[db], add=True)

    _issue_idx(0, src0, dst0, ew0, isem0)
    _wait_idx(0, src0, dst0, ew0, isem0)
    _gather(src0, rows0).start()
    _issue_idx(1, src1, dst1, ew1, isem1)

    def _pair(g, _):
        a = 2 * g
        _wait_idx(a + 1, src1, dst1, ew1, isem1)
        _gather(src0, rows0).wait()
        _gather(src1, rows1).start()
        _scale(rows0, ew0)
        _scatter(rows0, dst0)
        _issue_idx(a + 2, src0, dst0, ew0, isem0)
        _gather(src1, rows1).wait()
        _scale(rows1, ew1)
        _wait_idx(a + 2, src0, dst0, ew0, isem0)
        _gather(src0, rows0).start()
        _scatter(rows1, dst1)
        _issue_idx(a + 3, src1, dst1, ew1, isem1)
        return ()

    lax.fori_loop(0, DCHUNK // 2 - 1, _pair, ())
    _wait_idx(DCHUNK - 1, src1, dst1, ew1, isem1)
    _gather(src0, rows0).wait()
    _gather(src1, rows1).start()
    _scale(rows0, ew0)
    _scatter(rows0, dst0)
    _gather(src1, rows1).wait()
    _scale(rows1, ew1)
    _scatter(rows1, dst1)

    plsc.subcore_barrier()
    pltpu.sync_copy(acc_sh.at[pl.ds(sid * RPT, RPT)],
                    out_hbm.at[cid, pl.ds(sid * RPT, RPT)])


RB = 2000
GRID = N // RB


def _mm_body(x_ref, w_ref, o_ref):
    o_ref[...] = lax.dot_general(x_ref[...], w_ref[...],
                                 (((1,), (0,)), ((), ())),
                                 preferred_element_type=jnp.float32,
                                 precision=_HI)


def _mm(x, w):
    return pl.pallas_call(
        _mm_body,
        grid=(GRID,),
        in_specs=[pl.BlockSpec((RB, D), lambda i: (i, 0)),
                  pl.BlockSpec((D, D), lambda i: (0, 0))],
        out_specs=pl.BlockSpec((RB, D), lambda i: (i, 0)),
        out_shape=jax.ShapeDtypeStruct((N, D), jnp.float32),
    )(x, w)


def _split_store(o_ref, full):
    o_ref[0] = full[:, :DH]
    o_ref[1] = full[:, DH:]


def _prep_body(h_ref, dinv_ref, hs_ref):
    _split_store(hs_ref, h_ref[...] * dinv_ref[...])


def _prep(h, dinv):
    return pl.pallas_call(
        _prep_body,
        grid=(GRID,),
        in_specs=[pl.BlockSpec((RB, D), lambda i: (i, 0)),
                  pl.BlockSpec((RB, 1), lambda i: (i, 0))],
        out_specs=pl.BlockSpec((NC, RB, DH), lambda i: (0, i, 0)),
        out_shape=jax.ShapeDtypeStruct((NC, NPAD, DH), jnp.float32),
    )(h, dinv)


def _elu(t):
    return jnp.where(t > 0, t, jnp.exp(t) - 1.0)


def _comb_pre(a_ref, hs_ref, dinv_ref, b_ref):
    a = a_ref[...]
    hs = hs_ref[...]
    acc = jnp.concatenate([a[0], a[1]], axis=1)
    hsf = jnp.concatenate([hs[0], hs[1]], axis=1)
    return _elu(dinv_ref[...] * (acc + hsf) + b_ref[...])


def _comb1_body(a_ref, hs_ref, dinv_ref, b_ref, w_ref, o_ref):
    t = _comb_pre(a_ref, hs_ref, dinv_ref, b_ref)
    h2 = dinv_ref[...] * lax.dot_general(t, w_ref[...],
                                         (((1,), (0,)), ((), ())),
                                         preferred_element_type=jnp.float32,
                                         precision=_HI)
    _split_store(o_ref, h2)


def _comb1(acc, hs, dinv, b, w):
    return pl.pallas_call(
        _comb1_body,
        grid=(GRID,),
        in_specs=[pl.BlockSpec((NC, RB, DH), lambda i: (0, i, 0)),
                  pl.BlockSpec((NC, RB, DH), lambda i: (0, i, 0)),
                  pl.BlockSpec((RB, 1), lambda i: (i, 0)),
                  pl.BlockSpec((1, D), lambda i: (0, 0)),
                  pl.BlockSpec((D, D), lambda i: (0, 0))],
        out_specs=pl.BlockSpec((NC, RB, DH), lambda i: (0, i, 0)),
        out_shape=jax.ShapeDtypeStruct((NC, NPAD, DH), jnp.float32),
    )(acc, hs, dinv, b, w)


def _comb2_body(a_ref, hs_ref, dinv_ref, b_ref, o_ref):
    o_ref[...] = _comb_pre(a_ref, hs_ref, dinv_ref, b_ref)


def _comb2(acc, hs, dinv, b):
    return pl.pallas_call(
        _comb2_body,
        grid=(GRID,),
        in_specs=[pl.BlockSpec((NC, RB, DH), lambda i: (0, i, 0)),
                  pl.BlockSpec((NC, RB, DH), lambda i: (0, i, 0)),
                  pl.BlockSpec((RB, 1), lambda i: (i, 0)),
                  pl.BlockSpec((1, D), lambda i: (0, 0))],
        out_specs=pl.BlockSpec((RB, D), lambda i: (i, 0)),
        out_shape=jax.ShapeDtypeStruct((N, D), jnp.float32),
    )(acc, hs, dinv, b)


KB = 2000
KGRID = INPUT // KB


def _fc_body(rt_ref, w1_ref, b1_ref, w2_ref, b2_ref, o_ref, acc_ref):
    i = pl.program_id(0)

    @pl.when(i == 0)
    def _():
        acc_ref[...] = jnp.zeros_like(acc_ref)

    acc_ref[...] += lax.dot_general(rt_ref[...], w1_ref[...],
                                    (((0,), (0,)), ((), ())),
                                    preferred_element_type=jnp.float32,
                                    precision=_HI)

    @pl.when(i == KGRID - 1)
    def _():
        z = _elu(acc_ref[...] + b1_ref[...])
        y = lax.dot_general(z, w2_ref[...], (((1,), (0,)), ((), ())),
                            preferred_element_type=jnp.float32,
                            precision=_HI) + b2_ref[...]
        o_ref[...] = _elu(y)


def _fc(rt, w1, b1, w2, b2):
    return pl.pallas_call(
        _fc_body,
        grid=(KGRID,),
        in_specs=[pl.BlockSpec((KB, D), lambda i: (i, 0)),
                  pl.BlockSpec((KB, FFN), lambda i: (i, 0)),
                  pl.BlockSpec((1, FFN), lambda i: (0, 0)),
                  pl.BlockSpec((FFN, BOT), lambda i: (0, 0)),
                  pl.BlockSpec((1, BOT), lambda i: (0, 0))],
        out_specs=pl.BlockSpec((D, BOT), lambda i: (0, 0)),
        out_shape=jax.ShapeDtypeStruct((D, BOT), jnp.float32),
        scratch_shapes=[pltpu.VMEM((D, FFN), jnp.float32)],
        compiler_params=pltpu.CompilerParams(
            dimension_semantics=("arbitrary",)),
    )(rt, w1, b1, w2, b2)


def kernel(x, edge_index, edge_weight, W1, b1, W2, b2,
           fc1_W, fc1_b, fc2_W, fc2_b):
    pad = EPAD - E
    srcp = jnp.concatenate([edge_index[0],
                            jnp.zeros((pad,), jnp.int32)])
    pad_dst = N + jax.lax.rem(jnp.arange(pad, dtype=jnp.int32),
                              jnp.int32(NPAD - N))
    dstp = jnp.concatenate([edge_index[1], pad_dst])
    ewp = jnp.concatenate([edge_weight, jnp.zeros((pad,), jnp.float32)])
    src3 = srcp.reshape(NS, DCHUNK, C)
    dst3 = dstp.reshape(NS, DCHUNK, C)
    ew3 = ewp.reshape(NS, DCHUNK, C)

    dinv_vec = _dinv_sc(dst3, ew3)
    h1 = _mm(x, W1)
    dinv = dinv_vec.reshape(NPAD, 1)
    h1s = _prep(h1, dinv)
    acc1 = _agg_sc(h1s, src3, dst3, ew3)
    h2s = _comb1(acc1, h1s, dinv, b1.reshape(1, D), W2)
    acc2 = _agg_sc(h2s, src3, dst3, ew3)
    out2 = _comb2(acc2, h2s, dinv, b2.reshape(1, D))
    rt = out2.reshape(D, INPUT).T
    return _fc(rt, fc1_W, fc1_b.reshape(1, FFN), fc2_W, fc2_b.reshape(1, BOT))
